# baseline (speedup 1.0000x reference)
import functools

import numpy as np
import jax
import jax.numpy as jnp
from jax.experimental import pallas as pl

_H = 1024
_NH = 16
_NKV = 4
_HD = 64
_E = 8
_TOPK = 2
_IM = 512
_ISH = 1024
_THETA = 500000.0
_EPS = 1e-06

_BS = 256
_BQ = 256
_BK = 256


def _rope_half(x, cos, sin):
    x1 = x[:, :32]
    x2 = x[:, 32:]
    return jnp.concatenate([x1 * cos - x2 * sin, x1 * sin + x2 * cos], axis=1)


def _pre_kernel(hid_ref, res_ref, wln_ref, wqkv_ref, res1_ref, qkv_ref):
    h = hid_ref[...] + res_ref[...]
    res1_ref[...] = h
    v = jnp.mean(h * h, axis=1, keepdims=True)
    ln = h * jax.lax.rsqrt(v + _EPS) * wln_ref[...]
    qkv_ref[...] = jnp.dot(ln, wqkv_ref[...], preferred_element_type=jnp.float32)


def _attn_kernel(cosq_ref, sinq_ref, cosk_ref, sink_ref, q_ref, k_ref, v_ref,
                 o_ref):
    i = pl.program_id(1)
    q = _rope_half(q_ref[0], cosq_ref[...], sinq_ref[...]) * (_HD ** -0.5)

    def body(j, carry):
        acc, m, l = carry
        kj = k_ref[0, pl.ds(j * _BK, _BK), :]
        ck = cosk_ref[pl.ds(j * _BK, _BK), :]
        sk = sink_ref[pl.ds(j * _BK, _BK), :]
        kr = _rope_half(kj, ck, sk)
        s = jax.lax.dot_general(q, kr, (((1,), (1,)), ((), ())),
                                preferred_element_type=jnp.float32)
        qpos = i * _BQ + jax.lax.broadcasted_iota(jnp.int32, (_BQ, _BK), 0)
        kpos = j * _BK + jax.lax.broadcasted_iota(jnp.int32, (_BQ, _BK), 1)
        s = jnp.where(qpos >= kpos, s, -1e30)
        m_new = jnp.maximum(m, jnp.max(s, axis=1, keepdims=True))
        alpha = jnp.exp(m - m_new)
        p = jnp.exp(s - m_new)
        vj = v_ref[0, pl.ds(j * _BK, _BK), :]
        l_new = l * alpha + jnp.sum(p, axis=1, keepdims=True)
        acc_new = acc * alpha + jnp.dot(p, vj,
                                        preferred_element_type=jnp.float32)
        return acc_new, m_new, l_new

    acc0 = jnp.zeros((_BQ, _HD), jnp.float32)
    m0 = jnp.full((_BQ, 1), -1e30, jnp.float32)
    l0 = jnp.zeros((_BQ, 1), jnp.float32)
    acc, m, l = jax.lax.fori_loop(0, i + 1, body, (acc0, m0, l0))
    o_ref[0] = acc / l


def _post_kernel(ctx_ref, res1_ref, wln_ref, wo_ref, gwt_ref, gb_ref,
                 res2_ref, h2_ref, we_ref):
    att = jnp.dot(ctx_ref[...], wo_ref[...], preferred_element_type=jnp.float32)
    r2 = att + res1_ref[...]
    res2_ref[...] = r2
    v = jnp.mean(r2 * r2, axis=1, keepdims=True)
    h2 = r2 * jax.lax.rsqrt(v + _EPS) * wln_ref[...]
    h2_ref[...] = h2
    logits = jnp.dot(h2, gwt_ref[...], preferred_element_type=jnp.float32)
    mx = jnp.max(logits, axis=1, keepdims=True)
    ex = jnp.exp(logits - mx)
    probs = ex / jnp.sum(ex, axis=1, keepdims=True)
    b = probs + gb_ref[...]
    idx = jax.lax.broadcasted_iota(jnp.int32, (_BS, _E), 1)
    m1 = jnp.max(b, axis=1, keepdims=True)
    a1 = jnp.min(jnp.where(b == m1, idx, _E), axis=1, keepdims=True)
    oh1 = idx == a1
    b2 = jnp.where(oh1, -1e30, b)
    m2 = jnp.max(b2, axis=1, keepdims=True)
    a2 = jnp.min(jnp.where(b2 == m2, idx, _E), axis=1, keepdims=True)
    sel = oh1 | (idx == a2)
    w = jnp.where(sel, probs, 0.0)
    we_ref[...] = w / jnp.sum(w, axis=1, keepdims=True)


def _shared_kernel(h2_ref, gu_ref, dn_ref, out_ref):
    g = jnp.dot(h2_ref[...], gu_ref[...], preferred_element_type=jnp.float32)
    g1 = g[:, :_ISH]
    g2 = g[:, _ISH:]
    a = g1 * jax.nn.sigmoid(g1) * g2
    out_ref[...] = jnp.dot(a, dn_ref[...], preferred_element_type=jnp.float32)


def _moe_kernel(h2_ref, we_ref, shared_ref, gu_ref, dn_ref, out_ref):
    e = pl.program_id(0)
    r = pl.program_id(1)
    g = jnp.dot(h2_ref[...], gu_ref[0], preferred_element_type=jnp.float32)
    g1 = g[:, :_IM]
    g2 = g[:, _IM:]
    a = g1 * jax.nn.sigmoid(g1) * g2
    xe = jnp.dot(a, dn_ref[0], preferred_element_type=jnp.float32)
    idx = jax.lax.broadcasted_iota(jnp.int32, (_BS, _E), 1)
    w = jnp.sum(we_ref[...] * (idx == e).astype(jnp.float32), axis=1,
                keepdims=True)
    contrib = w * xe
    rows = pl.ds(r * _BS, _BS)

    @pl.when(e == 0)
    def _():
        out_ref[rows, :] = shared_ref[...] + contrib

    @pl.when(e != 0)
    def _():
        out_ref[rows, :] = out_ref[rows, :] + contrib


def kernel(hidden_states, residual, w_in_ln, W_qkv, W_o, w_post_ln, gate_w,
           gate_bias, exp_gu, exp_dn, sh_gu, sh_dn, positions):
    T = hidden_states.shape[0]
    nq = _NH * _HD

    half = np.arange(0, _HD, 2)
    head_perm = np.concatenate([half, half + 1])
    qperm = (np.arange(_NH)[:, None] * _HD + head_perm[None, :]).reshape(-1)
    kperm = nq + (np.arange(_NKV)[:, None] * _HD + head_perm[None, :]).reshape(-1)
    vcols = np.arange(nq + _NKV * _HD, nq + 2 * _NKV * _HD)
    perm = np.concatenate([qperm, kperm, vcols])
    wqkv_p = W_qkv[:, perm]

    inv_freq = 1.0 / (_THETA ** (jnp.arange(0, _HD, 2, dtype=jnp.float32) / _HD))
    freqs = positions.astype(jnp.float32)[:, None] * inv_freq[None, :]
    cos_t = jnp.cos(freqs)
    sin_t = jnp.sin(freqs)

    nR = T // _BS

    res1, qkv = pl.pallas_call(
        _pre_kernel,
        grid=(nR,),
        in_specs=[
            pl.BlockSpec((_BS, _H), lambda r: (r, 0)),
            pl.BlockSpec((_BS, _H), lambda r: (r, 0)),
            pl.BlockSpec((1, _H), lambda r: (0, 0)),
            pl.BlockSpec((_H, (_NH + 2 * _NKV) * _HD), lambda r: (0, 0)),
        ],
        out_specs=[
            pl.BlockSpec((_BS, _H), lambda r: (r, 0)),
            pl.BlockSpec((_BS, (_NH + 2 * _NKV) * _HD), lambda r: (r, 0)),
        ],
        out_shape=[
            jax.ShapeDtypeStruct((T, _H), jnp.float32),
            jax.ShapeDtypeStruct((T, (_NH + 2 * _NKV) * _HD), jnp.float32),
        ],
    )(hidden_states, residual, w_in_ln.reshape(1, _H), wqkv_p)

    q = qkv[:, :nq].reshape(T, _NH, _HD).transpose(1, 0, 2)
    k = qkv[:, nq:nq + _NKV * _HD].reshape(T, _NKV, _HD).transpose(1, 0, 2)
    v = qkv[:, nq + _NKV * _HD:].reshape(T, _NKV, _HD).transpose(1, 0, 2)

    rep = _NH // _NKV
    ctx = pl.pallas_call(
        _attn_kernel,
        grid=(_NH, T // _BQ),
        in_specs=[
            pl.BlockSpec((_BQ, _HD // 2), lambda h, i: (i, 0)),
            pl.BlockSpec((_BQ, _HD // 2), lambda h, i: (i, 0)),
            pl.BlockSpec((T, _HD // 2), lambda h, i: (0, 0)),
            pl.BlockSpec((T, _HD // 2), lambda h, i: (0, 0)),
            pl.BlockSpec((1, _BQ, _HD), lambda h, i: (h, i, 0)),
            pl.BlockSpec((1, T, _HD), lambda h, i: (h // rep, 0, 0)),
            pl.BlockSpec((1, T, _HD), lambda h, i: (h // rep, 0, 0)),
        ],
        out_specs=pl.BlockSpec((1, _BQ, _HD), lambda h, i: (h, i, 0)),
        out_shape=jax.ShapeDtypeStruct((_NH, T, _HD), jnp.float32),
    )(cos_t, sin_t, cos_t, sin_t, q, k, v)

    ctx2 = ctx.transpose(1, 0, 2).reshape(T, nq)

    res2, h2, we = pl.pallas_call(
        _post_kernel,
        grid=(nR,),
        in_specs=[
            pl.BlockSpec((_BS, nq), lambda r: (r, 0)),
            pl.BlockSpec((_BS, _H), lambda r: (r, 0)),
            pl.BlockSpec((1, _H), lambda r: (0, 0)),
            pl.BlockSpec((nq, _H), lambda r: (0, 0)),
            pl.BlockSpec((_H, _E), lambda r: (0, 0)),
            pl.BlockSpec((1, _E), lambda r: (0, 0)),
        ],
        out_specs=[
            pl.BlockSpec((_BS, _H), lambda r: (r, 0)),
            pl.BlockSpec((_BS, _H), lambda r: (r, 0)),
            pl.BlockSpec((_BS, _E), lambda r: (r, 0)),
        ],
        out_shape=[
            jax.ShapeDtypeStruct((T, _H), jnp.float32),
            jax.ShapeDtypeStruct((T, _H), jnp.float32),
            jax.ShapeDtypeStruct((T, _E), jnp.float32),
        ],
    )(ctx2, res1, w_post_ln.reshape(1, _H), W_o, gate_w.T, gate_bias)

    shared = pl.pallas_call(
        _shared_kernel,
        grid=(nR,),
        in_specs=[
            pl.BlockSpec((_BS, _H), lambda r: (r, 0)),
            pl.BlockSpec((_H, 2 * _ISH), lambda r: (0, 0)),
            pl.BlockSpec((_ISH, _H), lambda r: (0, 0)),
        ],
        out_specs=pl.BlockSpec((_BS, _H), lambda r: (r, 0)),
        out_shape=jax.ShapeDtypeStruct((T, _H), jnp.float32),
    )(h2, sh_gu, sh_dn)

    h_out = pl.pallas_call(
        _moe_kernel,
        grid=(_E, nR),
        in_specs=[
            pl.BlockSpec((_BS, _H), lambda e, r: (r, 0)),
            pl.BlockSpec((_BS, _E), lambda e, r: (r, 0)),
            pl.BlockSpec((_BS, _H), lambda e, r: (r, 0)),
            pl.BlockSpec((1, _H, 2 * _IM), lambda e, r: (e, 0, 0)),
            pl.BlockSpec((1, _IM, _H), lambda e, r: (e, 0, 0)),
        ],
        out_specs=pl.BlockSpec((T, _H), lambda e, r: (0, 0)),
        out_shape=jax.ShapeDtypeStruct((T, _H), jnp.float32),
    )(h2, we, shared, exp_gu, exp_dn)

    return h_out, res2
